# baseline (speedup 1.0000x reference)
import functools

import jax
import jax.numpy as jnp
from jax import lax
from jax.experimental import pallas as pl
from jax.experimental.pallas import tpu as pltpu
from jax.experimental.pallas import tpu_sc as plsc

N = 10000
E = 320000
NC, NS = 2, 16
NW = NC * NS
CHUNK = 64
EPW = 10240
CPW = EPW // CHUNK
E_PAD = NW * EPW
NROW = 10240
RPW = NROW // NS
DW = 128

_MESH = plsc.VectorSubcoreMesh(
    core_axis_name="c", subcore_axis_name="s", num_cores=NC, num_subcores=NS
)


def _deg_body(dst_hbm, ones_hbm, zeros_hbm, out_hbm, dst_c, ones_v, acc):
    cid = lax.axis_index("c")
    sid = lax.axis_index("s")
    wid = cid * NS + sid
    pltpu.sync_copy(zeros_hbm, acc.at[pl.ds(sid * RPW, RPW)])
    pltpu.sync_copy(ones_hbm, ones_v)
    plsc.subcore_barrier()

    def body(i, c):
        base = wid * EPW + i * CHUNK
        pltpu.sync_copy(dst_hbm.at[pl.ds(base, CHUNK)], dst_c)
        pltpu.sync_copy(ones_v, acc.at[dst_c], add=True)
        return c

    lax.fori_loop(0, CPW, body, 0)

    plsc.subcore_barrier()
    pltpu.sync_copy(
        acc.at[pl.ds(sid * RPW, RPW)], out_hbm.at[cid, pl.ds(sid * RPW, RPW)]
    )


_deg_kernel = functools.partial(
    pl.kernel,
    out_type=jax.ShapeDtypeStruct((NC, NROW, DW), jnp.float32),
    mesh=_MESH,
    scratch_types=[
        pltpu.VMEM((CHUNK,), jnp.int32),
        pltpu.VMEM((CHUNK, DW), jnp.float32),
        pltpu.VMEM_SHARED((NROW, DW), jnp.float32),
    ],
)(_deg_body)


_DEPTH = 4


def _prop_body(g_hbm, src_hbm, dst_hbm, zeros_hbm, out_hbm,
               src_all, dbufs, rbufs, isems, rsems, acc):
    cid = lax.axis_index("c")
    sid = lax.axis_index("s")
    wid = cid * NS + sid
    base_w = wid * EPW
    pltpu.sync_copy(zeros_hbm, acc.at[pl.ds(sid * RPW, RPW)])
    pltpu.sync_copy(src_hbm.at[pl.ds(base_w, EPW)], src_all)
    plsc.subcore_barrier()

    def src_at(i):
        return src_all.at[pl.ds(i * CHUNK, CHUNK)]

    def dst_at(i):
        return dst_hbm.at[pl.ds(base_w + i * CHUNK, CHUNK)]

    def body(k, c):
        i = k * _DEPTH
        gs = [
            pltpu.async_copy(g_hbm.at[src_at(i + j)], rbufs[j], rsems[j])
            for j in range(_DEPTH)
        ]
        ds = [
            pltpu.async_copy(dst_at(i + j), dbufs[j], isems[j])
            for j in range(_DEPTH)
        ]
        for j in range(_DEPTH):
            gs[j].wait()
            ds[j].wait()
            pltpu.sync_copy(rbufs[j], acc.at[dbufs[j]], add=True)
        return c

    lax.fori_loop(0, CPW // _DEPTH, body, 0)

    plsc.subcore_barrier()
    pltpu.sync_copy(
        acc.at[pl.ds(sid * RPW, RPW)], out_hbm.at[cid, pl.ds(sid * RPW, RPW)]
    )


def _make_prop(D):
    return functools.partial(
        pl.kernel,
        out_type=jax.ShapeDtypeStruct((NC, NROW, D), jnp.float32),
        mesh=_MESH,
        scratch_types=[
            pltpu.VMEM((EPW,), jnp.int32),
            [pltpu.VMEM((CHUNK,), jnp.int32) for _ in range(_DEPTH)],
            [pltpu.VMEM((CHUNK, D), jnp.float32) for _ in range(_DEPTH)],
            [pltpu.SemaphoreType.DMA for _ in range(_DEPTH)],
            [pltpu.SemaphoreType.DMA for _ in range(_DEPTH)],
            pltpu.VMEM_SHARED((NROW, D), jnp.float32),
        ],
    )(_prop_body)


_prop128 = _make_prop(128)


def _dense0_body(x_ref, w_ref, degp_ref, g_ref, dis_ref):
    d = degp_ref[0, 0:N, 0:1] + degp_ref[1, 0:N, 0:1] + 1.0
    dis = lax.rsqrt(d)
    h = jnp.dot(x_ref[...], w_ref[...], preferred_element_type=jnp.float32)
    g_ref[...] = h * dis
    dis_ref[...] = dis


_dense0 = pl.pallas_call(
    _dense0_body,
    out_shape=[
        jax.ShapeDtypeStruct((N, 128), jnp.float32),
        jax.ShapeDtypeStruct((N, 1), jnp.float32),
    ],
)


def _dense_mid_body(p_ref, g_ref, dis_ref, b_ref, w_ref, out_ref):
    dis = dis_ref[...]
    z = (p_ref[0, 0:N, :] + p_ref[1, 0:N, :] + g_ref[...]) * dis + b_ref[...]
    a = jnp.maximum(z, 0.0)
    out_ref[...] = jnp.dot(a, w_ref[...], preferred_element_type=jnp.float32) * dis


def _make_mid(Dout):
    return pl.pallas_call(
        _dense_mid_body,
        out_shape=jax.ShapeDtypeStruct((N, Dout), jnp.float32),
    )


_mid128 = _make_mid(128)


def _final_body(p_ref, g_ref, dis_ref, b_ref, out_ref):
    dis = dis_ref[...]
    z = (p_ref[0, 0:N, :] + p_ref[1, 0:N, :] + g_ref[...]) * dis + b_ref[...]
    a = jnp.maximum(z, 0.0)
    col = lax.broadcasted_iota(jnp.int32, (N, 128), 1)
    mask = col < 40
    am = jnp.where(mask, a, -jnp.inf)
    m = jnp.max(am, axis=1, keepdims=True)
    e = jnp.where(mask, jnp.exp(a - m), 0.0)
    lse = jnp.log(jnp.sum(e, axis=1, keepdims=True)) + m
    res = a - lse
    out_ref[...] = res[:, :40]


_final = pl.pallas_call(
    _final_body,
    out_shape=jax.ShapeDtypeStruct((N, 40), jnp.float32),
)


def kernel(x, edge_index, W0, b0, W1, b1, W2, b2, W3, b3):
    npad = E_PAD - E
    src = jnp.concatenate([edge_index[0], jnp.zeros((npad,), jnp.int32)])
    dst = jnp.concatenate([edge_index[1], jnp.full((npad,), N, jnp.int32)])
    ones_dw = jnp.ones((CHUNK, DW), jnp.float32)
    zeros_dw = jnp.zeros((RPW, DW), jnp.float32)
    zeros128 = jnp.zeros((RPW, 128), jnp.float32)
    W3p = jnp.pad(W3, ((0, 0), (0, 88)))
    b3p = jnp.pad(b3, (0, 88)).reshape(1, 128)

    degp = _deg_kernel(dst, ones_dw, zeros_dw)
    g0, dis = _dense0(x, W0, degp)
    p1 = _prop128(g0, src, dst, zeros128)
    g1 = _mid128(p1, g0, dis, b0.reshape(1, 128), W1)
    p2 = _prop128(g1, src, dst, zeros128)
    g2 = _mid128(p2, g1, dis, b1.reshape(1, 128), W2)
    p3 = _prop128(g2, src, dst, zeros128)
    g3 = _mid128(p3, g2, dis, b2.reshape(1, 128), W3p)
    p4 = _prop128(g3, src, dst, zeros128)
    return _final(p4, g3, dis, b3p)


# speedup vs baseline: 1.2220x; 1.2220x over previous
import functools

import jax
import jax.numpy as jnp
from jax import lax
from jax.experimental import pallas as pl
from jax.experimental.pallas import tpu as pltpu
from jax.experimental.pallas import tpu_sc as plsc

N = 10000
E = 320000
NC, NS = 2, 16
NW = NC * NS
CHUNK = 128
EPW = 10240
CPW = EPW // CHUNK
E_PAD = NW * EPW
NROW = 10240
RPW = NROW // NS
DW = 128

_MESH = plsc.VectorSubcoreMesh(
    core_axis_name="c", subcore_axis_name="s", num_cores=NC, num_subcores=NS
)


def _deg_body(dst_hbm, ones_hbm, zeros_hbm, out_hbm, dst_c, ones_v, acc):
    cid = lax.axis_index("c")
    sid = lax.axis_index("s")
    wid = cid * NS + sid
    pltpu.sync_copy(zeros_hbm, acc.at[pl.ds(sid * RPW, RPW)])
    pltpu.sync_copy(ones_hbm, ones_v)
    plsc.subcore_barrier()

    def body(i, c):
        base = wid * EPW + i * CHUNK
        pltpu.sync_copy(dst_hbm.at[pl.ds(base, CHUNK)], dst_c)
        pltpu.sync_copy(ones_v, acc.at[dst_c], add=True)
        return c

    lax.fori_loop(0, CPW, body, 0)

    plsc.subcore_barrier()
    pltpu.sync_copy(
        acc.at[pl.ds(sid * RPW, RPW)], out_hbm.at[cid, pl.ds(sid * RPW, RPW)]
    )


_deg_kernel = functools.partial(
    pl.kernel,
    out_type=jax.ShapeDtypeStruct((NC, NROW, DW), jnp.float32),
    mesh=_MESH,
    scratch_types=[
        pltpu.VMEM((CHUNK,), jnp.int32),
        pltpu.VMEM((CHUNK, DW), jnp.float32),
        pltpu.VMEM_SHARED((NROW, DW), jnp.float32),
    ],
)(_deg_body)


_DEPTH = 2


EPW_A = 15360
EPW_B = 2 * EPW - EPW_A
CPW_A = EPW_A // CHUNK
CPW_B = EPW_B // CHUNK


def _prop_body(g_hbm, src_hbm, dst_hbm, zeros_hbm, out_hbm,
               src_all, dbufs, rbufs, isems, rsems, acc):
    cid = lax.axis_index("c")
    sid = lax.axis_index("s")
    base_w = jnp.where(cid == 0, sid * EPW_A, NS * EPW_A + sid * EPW_B)
    cpw_w = jnp.where(cid == 0, CPW_A, CPW_B)
    pltpu.sync_copy(zeros_hbm, acc.at[pl.ds(sid * RPW, RPW)])

    @pl.when(cid == 0)
    def _():
        pltpu.sync_copy(src_hbm.at[pl.ds(base_w, EPW_A)], src_all.at[pl.ds(0, EPW_A)])

    @pl.when(cid == 1)
    def _():
        pltpu.sync_copy(src_hbm.at[pl.ds(base_w, EPW_B)], src_all.at[pl.ds(0, EPW_B)])

    plsc.subcore_barrier()

    def src_at(i):
        return src_all.at[pl.ds(i * CHUNK, CHUNK)]

    def dst_at(i):
        return dst_hbm.at[pl.ds(base_w + i * CHUNK, CHUNK)]

    def body(k, c):
        i = k * _DEPTH
        gs = [
            pltpu.async_copy(g_hbm.at[src_at(i + j)], rbufs[j], rsems[j])
            for j in range(_DEPTH)
        ]
        ds = [
            pltpu.async_copy(dst_at(i + j), dbufs[j], isems[j])
            for j in range(_DEPTH)
        ]
        for j in range(_DEPTH):
            gs[j].wait()
            ds[j].wait()
            pltpu.sync_copy(rbufs[j], acc.at[dbufs[j]], add=True)
        return c

    lax.fori_loop(0, cpw_w // _DEPTH, body, 0)

    plsc.subcore_barrier()
    pltpu.sync_copy(
        acc.at[pl.ds(sid * RPW, RPW)], out_hbm.at[cid, pl.ds(sid * RPW, RPW)]
    )


def _make_prop(D):
    return functools.partial(
        pl.kernel,
        out_type=jax.ShapeDtypeStruct((NC, NROW, D), jnp.float32),
        mesh=_MESH,
        scratch_types=[
            pltpu.VMEM((EPW_A,), jnp.int32),
            [pltpu.VMEM((CHUNK,), jnp.int32) for _ in range(_DEPTH)],
            [pltpu.VMEM((CHUNK, D), jnp.float32) for _ in range(_DEPTH)],
            [pltpu.SemaphoreType.DMA for _ in range(_DEPTH)],
            [pltpu.SemaphoreType.DMA for _ in range(_DEPTH)],
            pltpu.VMEM_SHARED((NROW, D), jnp.float32),
        ],
    )(_prop_body)


_prop128 = _make_prop(128)


def _dense0_body(x_ref, w_ref, degp_ref, g_ref, dis_ref):
    d = degp_ref[0, 0:N, 0:1] + degp_ref[1, 0:N, 0:1] + 1.0
    dis = lax.rsqrt(d)
    h = jnp.dot(x_ref[...], w_ref[...], preferred_element_type=jnp.float32)
    g_ref[...] = h * dis
    dis_ref[...] = dis


_dense0 = pl.pallas_call(
    _dense0_body,
    out_shape=[
        jax.ShapeDtypeStruct((N, 128), jnp.float32),
        jax.ShapeDtypeStruct((N, 1), jnp.float32),
    ],
)


def _dense_mid_body(p_ref, g_ref, dis_ref, b_ref, w_ref, out_ref):
    dis = dis_ref[...]
    z = (p_ref[0, 0:N, :] + p_ref[1, 0:N, :] + g_ref[...]) * dis + b_ref[...]
    a = jnp.maximum(z, 0.0)
    out_ref[...] = jnp.dot(a, w_ref[...], preferred_element_type=jnp.float32) * dis


def _make_mid(Dout):
    return pl.pallas_call(
        _dense_mid_body,
        out_shape=jax.ShapeDtypeStruct((N, Dout), jnp.float32),
    )


_mid128 = _make_mid(128)


def _final_body(p_ref, g_ref, dis_ref, b_ref, out_ref):
    dis = dis_ref[...]
    z = (p_ref[0, 0:N, :] + p_ref[1, 0:N, :] + g_ref[...]) * dis + b_ref[...]
    a = jnp.maximum(z, 0.0)
    col = lax.broadcasted_iota(jnp.int32, (N, 128), 1)
    mask = col < 40
    am = jnp.where(mask, a, -jnp.inf)
    m = jnp.max(am, axis=1, keepdims=True)
    e = jnp.where(mask, jnp.exp(a - m), 0.0)
    lse = jnp.log(jnp.sum(e, axis=1, keepdims=True)) + m
    res = a - lse
    out_ref[...] = res[:, :40]


_final = pl.pallas_call(
    _final_body,
    out_shape=jax.ShapeDtypeStruct((N, 40), jnp.float32),
)


def kernel(x, edge_index, W0, b0, W1, b1, W2, b2, W3, b3):
    npad = E_PAD - E
    src = jnp.concatenate([edge_index[0], jnp.zeros((npad,), jnp.int32)])
    dst = jnp.concatenate([edge_index[1], jnp.full((npad,), N, jnp.int32)])
    ones_dw = jnp.ones((CHUNK, DW), jnp.float32)
    zeros_dw = jnp.zeros((RPW, DW), jnp.float32)
    zeros128 = jnp.zeros((RPW, 128), jnp.float32)
    W3p = jnp.pad(W3, ((0, 0), (0, 88)))
    b3p = jnp.pad(b3, (0, 88)).reshape(1, 128)

    degp = _deg_kernel(dst, ones_dw, zeros_dw)
    g0, dis = _dense0(x, W0, degp)
    p1 = _prop128(g0, src, dst, zeros128)
    g1 = _mid128(p1, g0, dis, b0.reshape(1, 128), W1)
    p2 = _prop128(g1, src, dst, zeros128)
    g2 = _mid128(p2, g1, dis, b1.reshape(1, 128), W2)
    p3 = _prop128(g2, src, dst, zeros128)
    g3 = _mid128(p3, g2, dis, b2.reshape(1, 128), W3p)
    p4 = _prop128(g3, src, dst, zeros128)
    return _final(p4, g3, dis, b3p)


# speedup vs baseline: 1.2269x; 1.0040x over previous
import functools

import jax
import jax.numpy as jnp
from jax import lax
from jax.experimental import pallas as pl
from jax.experimental.pallas import tpu as pltpu
from jax.experimental.pallas import tpu_sc as plsc

N = 10000
E = 320000
NC, NS = 2, 16
NW = NC * NS
CHUNK = 128
EPW = 10240
CPW = EPW // CHUNK
E_PAD = NW * EPW
NROW = 10112
RPW = NROW // NS
DW = 128

_MESH = plsc.VectorSubcoreMesh(
    core_axis_name="c", subcore_axis_name="s", num_cores=NC, num_subcores=NS
)


def _deg_body(dst_hbm, ones_hbm, zeros_hbm, out_hbm, dst_c, ones_v, acc):
    cid = lax.axis_index("c")
    sid = lax.axis_index("s")
    wid = cid * NS + sid
    pltpu.sync_copy(zeros_hbm, acc.at[pl.ds(sid * RPW, RPW)])
    pltpu.sync_copy(ones_hbm, ones_v)
    plsc.subcore_barrier()

    def body(i, c):
        base = wid * EPW + i * CHUNK
        pltpu.sync_copy(dst_hbm.at[pl.ds(base, CHUNK)], dst_c)
        pltpu.sync_copy(ones_v, acc.at[dst_c], add=True)
        return c

    lax.fori_loop(0, CPW, body, 0)

    plsc.subcore_barrier()
    pltpu.sync_copy(
        acc.at[pl.ds(sid * RPW, RPW)], out_hbm.at[cid, pl.ds(sid * RPW, RPW)]
    )


_deg_kernel = functools.partial(
    pl.kernel,
    out_type=jax.ShapeDtypeStruct((NC, NROW, DW), jnp.float32),
    mesh=_MESH,
    scratch_types=[
        pltpu.VMEM((CHUNK,), jnp.int32),
        pltpu.VMEM((CHUNK, DW), jnp.float32),
        pltpu.VMEM_SHARED((NROW, DW), jnp.float32),
    ],
)(_deg_body)


_DEPTH = 2


EPW_A = 17408
EPW_B = 2 * EPW - EPW_A
CPW_A = EPW_A // CHUNK
CPW_B = EPW_B // CHUNK
HEPW_A, HEPW_B = EPW_A // 2, EPW_B // 2
HCPW_A, HCPW_B = CPW_A // 2, CPW_B // 2


def _prop_body(g_hbm, src_hbm, dst_hbm, zeros_hbm, out_hbm,
               src_all, dbufs, rbufs, isems, rsems, acc):
    cid = lax.axis_index("c")
    sid = lax.axis_index("s")
    base_w = jnp.where(cid == 0, sid * EPW_A, NS * EPW_A + sid * EPW_B)
    hcpw_w = jnp.where(cid == 0, HCPW_A, HCPW_B)
    pltpu.sync_copy(zeros_hbm, acc.at[pl.ds(sid * RPW, RPW)])

    def src_at(i):
        return src_all.at[pl.ds(i * CHUNK, CHUNK)]

    for h in range(2):
        @pl.when(cid == 0)
        def _():
            pltpu.sync_copy(
                src_hbm.at[pl.ds(base_w + h * HEPW_A, HEPW_A)],
                src_all.at[pl.ds(0, HEPW_A)],
            )

        @pl.when(cid == 1)
        def _():
            pltpu.sync_copy(
                src_hbm.at[pl.ds(base_w + h * HEPW_B, HEPW_B)],
                src_all.at[pl.ds(0, HEPW_B)],
            )

        if h == 0:
            plsc.subcore_barrier()

        hoff = h * hcpw_w * CHUNK

        def dst_at(i):
            return dst_hbm.at[pl.ds(base_w + hoff + i * CHUNK, CHUNK)]

        def body(k, c):
            i = k * _DEPTH
            gs = [
                pltpu.async_copy(g_hbm.at[src_at(i + j)], rbufs[j], rsems[j])
                for j in range(_DEPTH)
            ]
            ds = [
                pltpu.async_copy(dst_at(i + j), dbufs[j], isems[j])
                for j in range(_DEPTH)
            ]
            for j in range(_DEPTH):
                gs[j].wait()
                ds[j].wait()
                pltpu.sync_copy(rbufs[j], acc.at[dbufs[j]], add=True)
            return c

        lax.fori_loop(0, hcpw_w // _DEPTH, body, 0)

    plsc.subcore_barrier()
    pltpu.sync_copy(
        acc.at[pl.ds(sid * RPW, RPW)], out_hbm.at[cid, pl.ds(sid * RPW, RPW)]
    )


def _make_prop(D):
    return functools.partial(
        pl.kernel,
        out_type=jax.ShapeDtypeStruct((NC, NROW, D), jnp.float32),
        mesh=_MESH,
        scratch_types=[
            pltpu.VMEM((HEPW_A,), jnp.int32),
            [pltpu.VMEM((CHUNK,), jnp.int32) for _ in range(_DEPTH)],
            [pltpu.VMEM((CHUNK, D), jnp.float32) for _ in range(_DEPTH)],
            [pltpu.SemaphoreType.DMA for _ in range(_DEPTH)],
            [pltpu.SemaphoreType.DMA for _ in range(_DEPTH)],
            pltpu.VMEM_SHARED((NROW, D), jnp.float32),
        ],
    )(_prop_body)


_prop128 = _make_prop(128)


def _dense0_body(x_ref, w_ref, degp_ref, g_ref, dis_ref):
    d = degp_ref[0, 0:N, 0:1] + degp_ref[1, 0:N, 0:1] + 1.0
    dis = lax.rsqrt(d)
    h = jnp.dot(x_ref[...], w_ref[...], preferred_element_type=jnp.float32)
    g_ref[...] = h * dis
    dis_ref[...] = dis


_dense0 = pl.pallas_call(
    _dense0_body,
    out_shape=[
        jax.ShapeDtypeStruct((N, 128), jnp.float32),
        jax.ShapeDtypeStruct((N, 1), jnp.float32),
    ],
)


def _dense_mid_body(p_ref, g_ref, dis_ref, b_ref, w_ref, out_ref):
    dis = dis_ref[...]
    z = (p_ref[0, 0:N, :] + p_ref[1, 0:N, :] + g_ref[...]) * dis + b_ref[...]
    a = jnp.maximum(z, 0.0)
    out_ref[...] = jnp.dot(a, w_ref[...], preferred_element_type=jnp.float32) * dis


def _make_mid(Dout):
    return pl.pallas_call(
        _dense_mid_body,
        out_shape=jax.ShapeDtypeStruct((N, Dout), jnp.float32),
    )


_mid128 = _make_mid(128)


def _final_body(p_ref, g_ref, dis_ref, b_ref, out_ref):
    dis = dis_ref[...]
    z = (p_ref[0, 0:N, :] + p_ref[1, 0:N, :] + g_ref[...]) * dis + b_ref[...]
    a = jnp.maximum(z, 0.0)
    col = lax.broadcasted_iota(jnp.int32, (N, 128), 1)
    mask = col < 40
    am = jnp.where(mask, a, -jnp.inf)
    m = jnp.max(am, axis=1, keepdims=True)
    e = jnp.where(mask, jnp.exp(a - m), 0.0)
    lse = jnp.log(jnp.sum(e, axis=1, keepdims=True)) + m
    res = a - lse
    out_ref[...] = res[:, :40]


_final = pl.pallas_call(
    _final_body,
    out_shape=jax.ShapeDtypeStruct((N, 40), jnp.float32),
)


def kernel(x, edge_index, W0, b0, W1, b1, W2, b2, W3, b3):
    npad = E_PAD - E
    src = jnp.concatenate([edge_index[0], jnp.zeros((npad,), jnp.int32)])
    dst = jnp.concatenate([edge_index[1], jnp.full((npad,), N, jnp.int32)])
    ones_dw = jnp.ones((CHUNK, DW), jnp.float32)
    zeros_dw = jnp.zeros((RPW, DW), jnp.float32)
    zeros128 = jnp.zeros((RPW, 128), jnp.float32)
    W3p = jnp.pad(W3, ((0, 0), (0, 88)))
    b3p = jnp.pad(b3, (0, 88)).reshape(1, 128)

    degp = _deg_kernel(dst, ones_dw, zeros_dw)
    g0, dis = _dense0(x, W0, degp)
    p1 = _prop128(g0, src, dst, zeros128)
    g1 = _mid128(p1, g0, dis, b0.reshape(1, 128), W1)
    p2 = _prop128(g1, src, dst, zeros128)
    g2 = _mid128(p2, g1, dis, b1.reshape(1, 128), W2)
    p3 = _prop128(g2, src, dst, zeros128)
    g3 = _mid128(p3, g2, dis, b2.reshape(1, 128), W3p)
    p4 = _prop128(g3, src, dst, zeros128)
    return _final(p4, g3, dis, b3p)
